# baseline (speedup 1.0000x reference)
import functools

import jax
import jax.numpy as jnp
from jax import lax
from jax.experimental import pallas as pl
from jax.experimental.pallas import tpu as pltpu
from jax.experimental.pallas import tpu_sc as plsc

N = 100000
E = 3200000
D_EDGE = 16
D_NODE = 128
H = 128

NC = 2
NS = 16
NW = NC * NS
G = 128
NGROUPS = E // G
UNITS = NGROUPS // 8
UPW = UNITS // NW
EXTRA_U = UNITS - UPW * NW
CH = 4
RPT = 6256
RPT_LAST = N - (NS - 1) * RPT
ZROWS = CH * G


def _sc_segment_sum(col, edge_attr):
    mesh = plsc.VectorSubcoreMesh(
        core_axis_name="c", subcore_axis_name="s", num_cores=NC, num_subcores=NS
    )

    @functools.partial(
        pl.kernel,
        mesh=mesh,
        compiler_params=pltpu.CompilerParams(use_tc_tiling_on_sc=False),
        out_type=jax.ShapeDtypeStruct((NC, N, D_EDGE), jnp.float32),
        scratch_types=[
            pltpu.VMEM((2, CH, G), jnp.int32),
            pltpu.VMEM((2, ZROWS, D_EDGE), jnp.float32),
            pltpu.VMEM_SHARED((N, D_EDGE), jnp.float32),
            pltpu.SemaphoreType.DMA,
            pltpu.SemaphoreType.DMA,
            pltpu.SemaphoreType.DMA,
        ],
    )
    def scatter_kernel(col_hbm, ea_hbm, out_hbm, idx_v, rows_v, acc, lsem0, lsem1, ssem):
        c = lax.axis_index("c")
        s = lax.axis_index("s")
        wid = c * NS + s
        lsems = [lsem0, lsem1]

        @pl.loop(0, ZROWS)
        def _(i):
            rows_v[0, i] = jnp.zeros((D_EDGE,), jnp.float32)

        zbase = s * RPT
        nz = RPT // ZROWS
        for t in range(nz):
            pltpu.sync_copy(rows_v.at[0], acc.at[pl.ds(zbase + t * ZROWS, ZROWS)])

        @pl.when(s < NS - 1)
        def _():
            pltpu.sync_copy(
                rows_v.at[0, pl.ds(0, RPT - nz * ZROWS)],
                acc.at[pl.ds(zbase + nz * ZROWS, RPT - nz * ZROWS)],
            )

        @pl.when(s == NS - 1)
        def _():
            pltpu.sync_copy(
                rows_v.at[0, pl.ds(0, RPT_LAST - nz * ZROWS)],
                acc.at[pl.ds(zbase + nz * ZROWS, RPT_LAST - nz * ZROWS)],
            )

        plsc.subcore_barrier()

        u0 = wid * UPW + jnp.minimum(wid, EXTRA_U)
        gq = u0 * 8
        nch = 2 * (UPW + (wid < EXTRA_U).astype(jnp.int32))

        def issue_load(ci, b):
            g = gq + ci * CH
            pltpu.async_copy(
                ea_hbm.at[pl.ds(g * G, CH * G)], rows_v.at[b], lsems[b]
            )
            pltpu.async_copy(col_hbm.at[0, pl.ds(g, CH)], idx_v.at[b], lsems[b])

        def wait_load(ci, b):
            g = gq + ci * CH
            pltpu.make_async_copy(
                ea_hbm.at[pl.ds(g * G, CH * G)], rows_v.at[b], lsems[b]
            ).wait()
            pltpu.make_async_copy(
                col_hbm.at[0, pl.ds(g, CH)], idx_v.at[b], lsems[b]
            ).wait()

        def do_scatter(b):
            descs = [
                pltpu.async_copy(
                    rows_v.at[b, pl.ds(j * G, G)],
                    acc.at[idx_v.at[b, j]],
                    ssem,
                    add=True,
                )
                for j in range(CH)
            ]
            for dsc in descs:
                dsc.wait()

        issue_load(0, 0)

        @pl.loop(0, nch, step=2)
        def _(ci0):
            for b in range(2):
                ci = ci0 + b

                @pl.when(ci + 1 < nch)
                def _():
                    issue_load(ci + 1, 1 - b)

                wait_load(ci, b)
                do_scatter(b)

        plsc.subcore_barrier()

        @pl.when(s < NS - 1)
        def _():
            pltpu.sync_copy(
                acc.at[pl.ds(zbase, RPT)], out_hbm.at[c, pl.ds(zbase, RPT)]
            )

        @pl.when(s == NS - 1)
        def _():
            pltpu.sync_copy(
                acc.at[pl.ds(zbase, RPT_LAST)],
                out_hbm.at[c, pl.ds(zbase, RPT_LAST)],
            )

    return scatter_kernel(col, edge_attr)


BLKE = E


def _pack_body(col_ref, o_ref):
    o_ref[...] = col_ref[...].reshape(1, BLKE // G, G)


def _pack_col(col):
    return pl.pallas_call(
        _pack_body,
        grid=(E // BLKE,),
        in_specs=[pl.BlockSpec((BLKE,), lambda i: (i,))],
        out_specs=pl.BlockSpec((1, BLKE // G, G), lambda i: (0, i, 0)),
        out_shape=jax.ShapeDtypeStruct((1, NGROUPS, G), jnp.int32),
    )(col)


BLK = 2000


def _mlp_body(x_ref, p_ref, w0a, w0b, b0, w1, b1, w2, b2, gam, bet, o_ref):
    x = x_ref[...]
    a = p_ref[0] + p_ref[1]
    h = jnp.dot(x, w0a[...], preferred_element_type=jnp.float32)
    h += jnp.dot(a, w0b[...], preferred_element_type=jnp.float32)
    h = jnp.maximum(h + b0[...], 0.0)
    h = jnp.maximum(
        jnp.dot(h, w1[...], preferred_element_type=jnp.float32) + b1[...], 0.0
    )
    o = jnp.dot(h, w2[...], preferred_element_type=jnp.float32) + b2[...]
    mu = jnp.mean(o, axis=-1, keepdims=True)
    var = jnp.mean((o - mu) * (o - mu), axis=-1, keepdims=True)
    o_ref[...] = (o - mu) * lax.rsqrt(var + 1e-5) * gam[...] + bet[...]


def _mlp(node_attr, partials, w0a, w0b, b0, w1, b1, w2, b2, gam, bet):
    full = lambda shape: pl.BlockSpec(shape, lambda i: (0,) * len(shape))
    return pl.pallas_call(
        _mlp_body,
        grid=(N // BLK,),
        in_specs=[
            pl.BlockSpec((BLK, D_NODE), lambda i: (i, 0)),
            pl.BlockSpec((NC, BLK, D_EDGE), lambda i: (0, i, 0)),
            full((D_NODE, H)),
            full((D_EDGE, H)),
            full((1, H)),
            full((H, H)),
            full((1, H)),
            full((H, D_NODE)),
            full((1, D_NODE)),
            full((1, D_NODE)),
            full((1, D_NODE)),
        ],
        out_specs=pl.BlockSpec((BLK, D_NODE), lambda i: (i, 0)),
        out_shape=jax.ShapeDtypeStruct((N, D_NODE), jnp.float32),
    )(node_attr, partials, w0a, w0b, b0, w1, b1, w2, b2, gam, bet)


def kernel(node_attr, edge_attr, edge_index, W0, b0, W1, b1, W2, b2, ln_gamma, ln_beta):
    partials = _sc_segment_sum(_pack_col(edge_index[1]), edge_attr)
    row = lambda v: v.reshape(1, -1)
    return _mlp(
        node_attr,
        partials,
        W0[:D_NODE],
        W0[D_NODE:],
        row(b0),
        W1,
        row(b1),
        W2,
        row(b2),
        row(ln_gamma),
        row(ln_beta),
    )


# speedup vs baseline: 1.0054x; 1.0054x over previous
import functools

import jax
import jax.numpy as jnp
from jax import lax
from jax.experimental import pallas as pl
from jax.experimental.pallas import tpu as pltpu
from jax.experimental.pallas import tpu_sc as plsc

N = 100000
E = 3200000
D_EDGE = 16
D_NODE = 128
H = 128

NC = 2
NS = 16
NW = NC * NS
G = 128
NGROUPS = E // G
UNITS = NGROUPS // 8
UPW = UNITS // NW
EXTRA_U = UNITS - UPW * NW
CH = 4
RPT = 6256
RPT_LAST = N - (NS - 1) * RPT
ZROWS = CH * G


def _sc_segment_sum(col, edge_attr):
    mesh = plsc.VectorSubcoreMesh(
        core_axis_name="c", subcore_axis_name="s", num_cores=NC, num_subcores=NS
    )

    @functools.partial(
        pl.kernel,
        mesh=mesh,
        compiler_params=pltpu.CompilerParams(use_tc_tiling_on_sc=False),
        out_type=jax.ShapeDtypeStruct((NC, N, D_EDGE), jnp.float32),
        scratch_types=[
            pltpu.VMEM((2, CH, G), jnp.int32),
            pltpu.VMEM((2, CH * G // D_EDGE, D_EDGE), jnp.int32),
            pltpu.VMEM((2, ZROWS, D_EDGE), jnp.float32),
            pltpu.VMEM_SHARED((N, D_EDGE), jnp.float32),
            pltpu.SemaphoreType.DMA,
            pltpu.SemaphoreType.DMA,
            pltpu.SemaphoreType.DMA,
        ],
    )
    def scatter_kernel(
        col_hbm, ea_hbm, out_hbm, idx_v, idx16_v, rows_v, acc, lsem0, lsem1, ssem
    ):
        c = lax.axis_index("c")
        s = lax.axis_index("s")
        wid = c * NS + s
        lsems = [lsem0, lsem1]

        @pl.loop(0, ZROWS)
        def _(i):
            rows_v[0, i] = jnp.zeros((D_EDGE,), jnp.float32)

        zbase = s * RPT
        nz = RPT // ZROWS
        for t in range(nz):
            pltpu.sync_copy(rows_v.at[0], acc.at[pl.ds(zbase + t * ZROWS, ZROWS)])

        @pl.when(s < NS - 1)
        def _():
            pltpu.sync_copy(
                rows_v.at[0, pl.ds(0, RPT - nz * ZROWS)],
                acc.at[pl.ds(zbase + nz * ZROWS, RPT - nz * ZROWS)],
            )

        @pl.when(s == NS - 1)
        def _():
            pltpu.sync_copy(
                rows_v.at[0, pl.ds(0, RPT_LAST - nz * ZROWS)],
                acc.at[pl.ds(zbase + nz * ZROWS, RPT_LAST - nz * ZROWS)],
            )

        plsc.subcore_barrier()

        u0 = wid * UPW + jnp.minimum(wid, EXTRA_U)
        gq = u0 * 8
        nch = 2 * (UPW + (wid < EXTRA_U).astype(jnp.int32))

        def issue_load(ci, b):
            g = gq + ci * CH
            pltpu.async_copy(
                ea_hbm.at[pl.ds(g * G, CH * G)], rows_v.at[b], lsems[b]
            )
            pltpu.async_copy(
                col_hbm.at[pl.ds(g * (G // D_EDGE), CH * G // D_EDGE)],
                idx16_v.at[b],
                lsems[b],
            )

        def wait_load(ci, b):
            g = gq + ci * CH
            pltpu.make_async_copy(
                ea_hbm.at[pl.ds(g * G, CH * G)], rows_v.at[b], lsems[b]
            ).wait()
            pltpu.make_async_copy(
                col_hbm.at[pl.ds(g * (G // D_EDGE), CH * G // D_EDGE)],
                idx16_v.at[b],
                lsems[b],
            ).wait()

        def do_scatter(b):
            for j in range(CH):
                for k in range(G // D_EDGE):
                    idx_v[b, j, pl.ds(k * D_EDGE, D_EDGE)] = idx16_v[
                        b, j * (G // D_EDGE) + k
                    ]
            descs = [
                pltpu.async_copy(
                    rows_v.at[b, pl.ds(j * G, G)],
                    acc.at[idx_v.at[b, j]],
                    ssem,
                    add=True,
                )
                for j in range(CH)
            ]
            for dsc in descs:
                dsc.wait()

        issue_load(0, 0)

        @pl.loop(0, nch, step=2)
        def _(ci0):
            for b in range(2):
                ci = ci0 + b

                @pl.when(ci + 1 < nch)
                def _():
                    issue_load(ci + 1, 1 - b)

                wait_load(ci, b)
                do_scatter(b)

        plsc.subcore_barrier()

        @pl.when(s < NS - 1)
        def _():
            pltpu.sync_copy(
                acc.at[pl.ds(zbase, RPT)], out_hbm.at[c, pl.ds(zbase, RPT)]
            )

        @pl.when(s == NS - 1)
        def _():
            pltpu.sync_copy(
                acc.at[pl.ds(zbase, RPT_LAST)],
                out_hbm.at[c, pl.ds(zbase, RPT_LAST)],
            )

    return scatter_kernel(col, edge_attr)


BLK = 2000


def _mlp_body(x_ref, p_ref, w0a, w0b, b0, w1, b1, w2, b2, gam, bet, o_ref):
    x = x_ref[...]
    a = p_ref[0] + p_ref[1]
    h = jnp.dot(x, w0a[...], preferred_element_type=jnp.float32)
    h += jnp.dot(a, w0b[...], preferred_element_type=jnp.float32)
    h = jnp.maximum(h + b0[...], 0.0)
    h = jnp.maximum(
        jnp.dot(h, w1[...], preferred_element_type=jnp.float32) + b1[...], 0.0
    )
    o = jnp.dot(h, w2[...], preferred_element_type=jnp.float32) + b2[...]
    mu = jnp.mean(o, axis=-1, keepdims=True)
    var = jnp.mean((o - mu) * (o - mu), axis=-1, keepdims=True)
    o_ref[...] = (o - mu) * lax.rsqrt(var + 1e-5) * gam[...] + bet[...]


def _mlp(node_attr, partials, w0a, w0b, b0, w1, b1, w2, b2, gam, bet):
    full = lambda shape: pl.BlockSpec(shape, lambda i: (0,) * len(shape))
    return pl.pallas_call(
        _mlp_body,
        grid=(N // BLK,),
        in_specs=[
            pl.BlockSpec((BLK, D_NODE), lambda i: (i, 0)),
            pl.BlockSpec((NC, BLK, D_EDGE), lambda i: (0, i, 0)),
            full((D_NODE, H)),
            full((D_EDGE, H)),
            full((1, H)),
            full((H, H)),
            full((1, H)),
            full((H, D_NODE)),
            full((1, D_NODE)),
            full((1, D_NODE)),
            full((1, D_NODE)),
        ],
        out_specs=pl.BlockSpec((BLK, D_NODE), lambda i: (i, 0)),
        out_shape=jax.ShapeDtypeStruct((N, D_NODE), jnp.float32),
    )(node_attr, partials, w0a, w0b, b0, w1, b1, w2, b2, gam, bet)


def kernel(node_attr, edge_attr, edge_index, W0, b0, W1, b1, W2, b2, ln_gamma, ln_beta):
    partials = _sc_segment_sum(
        edge_index[1].reshape(E // D_EDGE, D_EDGE), edge_attr
    )
    row = lambda v: v.reshape(1, -1)
    return _mlp(
        node_attr,
        partials,
        W0[:D_NODE],
        W0[D_NODE:],
        row(b0),
        W1,
        row(b1),
        W2,
        row(b2),
        row(ln_gamma),
        row(ln_beta),
    )
